# baseline (speedup 1.0000x reference)
import functools

import jax
import jax.numpy as jnp
from jax import lax
from jax.experimental import pallas as pl
from jax.experimental.pallas import tpu as pltpu
from jax.experimental.pallas import tpu_sc as plsc

_NUM_CORES = 2
_NUM_SUBCORES = 16
_NUM_WORKERS = _NUM_CORES * _NUM_SUBCORES
_LANES = 16
_BBLK = 128


@functools.lru_cache(maxsize=None)
def _make_gather(batch, seq, d, vocab):
    assert batch == _NUM_WORKERS * _BBLK and d % 8 == 0 and seq % 8 == 0
    mesh = plsc.VectorSubcoreMesh(core_axis_name="c", subcore_axis_name="s")

    @functools.partial(
        pl.kernel,
        mesh=mesh,
        compiler_params=pltpu.CompilerParams(use_tc_tiling_on_sc=False,
                                             needs_layout_passes=False),
        out_type=jax.ShapeDtypeStruct(
            (seq, d // 8, batch // _BBLK, 8 * _BBLK), jnp.float32),
        scratch_types=[
            pltpu.VMEM((2, _BBLK), jnp.int32),
            pltpu.VMEM((2, _BBLK, 64), jnp.float32),
            pltpu.VMEM((2, d * _BBLK), jnp.float32),
            pltpu.SemaphoreType.DMA,
            pltpu.SemaphoreType.DMA,
            pltpu.SemaphoreType.DMA,
            pltpu.SemaphoreType.DMA,
        ],
    )
    def gather_kernel(table_hbm, xq_hbm, out_hbm, idxb, rowsb, transb,
                      g0, g1, w0, w1):
        gs = (g0, g1)
        ws = (w0, w1)
        wid = lax.axis_index("s") * _NUM_CORES + lax.axis_index("c")
        jvs = [jnp.arange(_LANES, dtype=jnp.int32) + (k * _LANES)
               for k in range(_BBLK // _LANES)]

        def fire_gather(s, p):
            pltpu.sync_copy(xq_hbm.at[s // 8, wid, s % 8], idxb.at[p])
            pltpu.async_copy(table_hbm.at[idxb.at[p]], rowsb.at[p], gs[p])

        def wait_gather(p):
            pltpu.make_async_copy(table_hbm.at[idxb.at[p]], rowsb.at[p],
                                  gs[p]).wait()

        def fire_write(s, p):
            for dt in range(d // 8):
                pltpu.async_copy(transb.at[p, pl.ds(dt * 8 * _BBLK, 8 * _BBLK)],
                                 out_hbm.at[s, dt, wid], ws[p])

        def wait_write(s, p):
            for dt in range(d // 8):
                pltpu.make_async_copy(
                    transb.at[p, pl.ds(dt * 8 * _BBLK, 8 * _BBLK)],
                    out_hbm.at[s, dt, wid], ws[p]).wait()

        iota = jnp.arange(_LANES, dtype=jnp.int32)
        rots = [jnp.mod(iota + r, _LANES) for r in range(_LANES)]
        scat = [rots[r] * _BBLK + iota for r in range(_LANES)]

        def transpose(p):
            rows = rowsb.at[p]
            trans = transb.at[p]

            def jtile(ji, carry):
                j0 = ji * _LANES
                jv = iota + j0
                for d0 in range(0, d, _LANES):
                    base = d0 * _BBLK + j0
                    dvs = [rots[r] + d0 for r in range(_LANES)]
                    vals = [plsc.load_gather(rows, [jv, dvs[q]])
                            for q in range(8)]
                    for q in range(8):
                        vals.append(plsc.load_gather(rows, [jv, dvs[q + 8]]))
                        plsc.store_scatter(trans, [scat[q] + base], vals[q])
                    for q in range(8, _LANES):
                        plsc.store_scatter(trans, [scat[q] + base], vals[q])
                return carry

            lax.fori_loop(0, _BBLK // _LANES, jtile, 0)

        fire_gather(0, 0)
        fire_gather(1, 1)
        wait_gather(0)
        transpose(0)
        fire_write(0, 0)
        fire_gather(2, 0)
        wait_gather(1)
        transpose(1)
        fire_write(1, 1)

        def body(j, carry):
            s = 2 * j
            fire_gather(s + 1, 1)
            wait_gather(0)
            wait_write(s - 2, 0)
            transpose(0)
            fire_write(s, 0)
            fire_gather(s + 2, 0)
            wait_gather(1)
            wait_write(s - 1, 1)
            transpose(1)
            fire_write(s + 1, 1)
            return carry

        lax.fori_loop(1, seq // 2 - 1, body, 0)

        s = seq - 2
        fire_gather(s + 1, 1)
        wait_gather(0)
        wait_write(s - 2, 0)
        transpose(0)
        fire_write(s, 0)
        wait_gather(1)
        wait_write(s - 1, 1)
        transpose(1)
        fire_write(s + 1, 1)
        wait_write(s, 0)
        wait_write(s + 1, 1)

    return gather_kernel


@jax.jit
def kernel(x, table):
    batch, seq = x.shape
    vocab, d = table.shape
    xi = x.astype(jnp.int32)
    xq = jnp.transpose(xi, (1, 0)).reshape(seq // 8, 8, batch // _BBLK,
                                           _BBLK).transpose(0, 2, 1, 3)
    out4 = _make_gather(batch, seq, d, vocab)(table, xq)
    out5 = out4.reshape(seq, d // 8, batch // _BBLK, 8, _BBLK)
    return jnp.transpose(out5, (2, 4, 0, 1, 3)).reshape(batch, seq, d)


# speedup vs baseline: 1.2175x; 1.2175x over previous
import functools

import jax
import jax.numpy as jnp
from jax import lax
from jax.experimental import pallas as pl
from jax.experimental.pallas import tpu as pltpu
from jax.experimental.pallas import tpu_sc as plsc

_NUM_CORES = 2
_NUM_SUBCORES = 16
_NUM_WORKERS = _NUM_CORES * _NUM_SUBCORES
_LANES = 16
_BBLK = 128


@functools.lru_cache(maxsize=None)
def _make_gather(batch, seq, d, vocab):
    assert batch == _NUM_WORKERS * _BBLK and d % 8 == 0 and seq % 8 == 0
    mesh = plsc.VectorSubcoreMesh(core_axis_name="c", subcore_axis_name="s")

    @functools.partial(
        pl.kernel,
        mesh=mesh,
        compiler_params=pltpu.CompilerParams(use_tc_tiling_on_sc=False,
                                             needs_layout_passes=False),
        out_type=jax.ShapeDtypeStruct(
            (seq, d // 8, batch // _BBLK, 8 * _BBLK), jnp.float32),
        scratch_types=[
            pltpu.VMEM((seq // 8, 8, _BBLK), jnp.int32),
            pltpu.VMEM((2, _BBLK, 64), jnp.float32),
            pltpu.VMEM((2, d * _BBLK), jnp.float32),
            pltpu.SemaphoreType.DMA,
            pltpu.SemaphoreType.DMA,
            pltpu.SemaphoreType.DMA,
            pltpu.SemaphoreType.DMA,
        ],
    )
    def gather_kernel(table_hbm, xq_hbm, out_hbm, idxall, rowsb, transb,
                      g0, g1, w0, w1):
        gs = (g0, g1)
        ws = (w0, w1)
        wid = lax.axis_index("s") * _NUM_CORES + lax.axis_index("c")

        for st in range(seq // 8):
            pltpu.sync_copy(xq_hbm.at[st, wid], idxall.at[st])

        def fire_gather(s, p):
            pltpu.async_copy(table_hbm.at[idxall.at[s // 8, s % 8]],
                             rowsb.at[p], gs[p])

        def wait_gather(p):
            pltpu.make_async_copy(table_hbm.at[idxall.at[0, 0]], rowsb.at[p],
                                  gs[p]).wait()

        def fire_write(s, p):
            for dt in range(d // 8):
                pltpu.async_copy(transb.at[p, pl.ds(dt * 8 * _BBLK, 8 * _BBLK)],
                                 out_hbm.at[s, dt, wid], ws[p])

        def wait_write(s, p):
            for dt in range(d // 8):
                pltpu.make_async_copy(
                    transb.at[p, pl.ds(dt * 8 * _BBLK, 8 * _BBLK)],
                    out_hbm.at[s, dt, wid], ws[p]).wait()

        iota = jnp.arange(_LANES, dtype=jnp.int32)
        rots = [jnp.mod(iota + r, _LANES) for r in range(_LANES)]
        scat = [rots[r] * _BBLK + iota for r in range(_LANES)]

        def transpose(p):
            rows = rowsb.at[p]
            trans = transb.at[p]

            def jtile(ji, carry):
                j0 = ji * _LANES
                jv = iota + j0
                for d0 in range(0, d, _LANES):
                    base = d0 * _BBLK + j0
                    for r0 in range(0, _LANES, 8):
                        dvs = [rots[r0 + q] + d0 for q in range(8)]
                        vals = [plsc.load_gather(rows, [jv, dv])
                                for dv in dvs]
                        for q, v in zip(range(8), vals):
                            plsc.store_scatter(
                                trans, [scat[r0 + q] + base], v)
                return carry

            lax.fori_loop(0, _BBLK // _LANES, jtile, 0)

        fire_gather(0, 0)
        fire_gather(1, 1)
        wait_gather(0)
        transpose(0)
        fire_write(0, 0)
        fire_gather(2, 0)
        wait_gather(1)
        transpose(1)
        fire_write(1, 1)

        def body(j, carry):
            s = 2 * j
            fire_gather(s + 1, 1)
            wait_gather(0)
            wait_write(s - 2, 0)
            transpose(0)
            fire_write(s, 0)
            fire_gather(s + 2, 0)
            wait_gather(1)
            wait_write(s - 1, 1)
            transpose(1)
            fire_write(s + 1, 1)
            return carry

        lax.fori_loop(1, seq // 2 - 1, body, 0)

        s = seq - 2
        fire_gather(s + 1, 1)
        wait_gather(0)
        wait_write(s - 2, 0)
        transpose(0)
        fire_write(s, 0)
        wait_gather(1)
        wait_write(s - 1, 1)
        transpose(1)
        fire_write(s + 1, 1)
        wait_write(s, 0)
        wait_write(s + 1, 1)

    return gather_kernel


@jax.jit
def kernel(x, table):
    batch, seq = x.shape
    vocab, d = table.shape
    xi = x.astype(jnp.int32)
    xq = jnp.transpose(xi, (1, 0)).reshape(seq // 8, 8, batch // _BBLK,
                                           _BBLK).transpose(0, 2, 1, 3)
    out4 = _make_gather(batch, seq, d, vocab)(table, xq)
    out5 = out4.reshape(seq, d // 8, batch // _BBLK, 8, _BBLK)
    return jnp.transpose(out5, (2, 4, 0, 1, 3)).reshape(batch, seq, d)


# speedup vs baseline: 1.2775x; 1.0493x over previous
import functools

import jax
import jax.numpy as jnp
from jax import lax
from jax.experimental import pallas as pl
from jax.experimental.pallas import tpu as pltpu
from jax.experimental.pallas import tpu_sc as plsc

_NUM_CORES = 2
_NUM_SUBCORES = 16
_NUM_WORKERS = _NUM_CORES * _NUM_SUBCORES
_LANES = 16
_BBLK = 128


@functools.lru_cache(maxsize=None)
def _make_gather(batch, seq, d, vocab):
    assert batch == _NUM_WORKERS * _BBLK and d % 8 == 0 and seq % 8 == 0
    mesh = plsc.VectorSubcoreMesh(core_axis_name="c", subcore_axis_name="s")

    @functools.partial(
        pl.kernel,
        mesh=mesh,
        compiler_params=pltpu.CompilerParams(use_tc_tiling_on_sc=False,
                                             needs_layout_passes=False),
        out_type=jax.ShapeDtypeStruct(
            (seq, d // 8, batch // _BBLK, 8 * _BBLK), jnp.float32),
        scratch_types=[
            pltpu.VMEM((seq // 8, 8, _BBLK), jnp.int32),
            pltpu.VMEM((2, _BBLK, 64), jnp.float32),
            pltpu.VMEM((2, d * _BBLK), jnp.float32),
            pltpu.SemaphoreType.DMA,
            pltpu.SemaphoreType.DMA,
            pltpu.SemaphoreType.DMA,
            pltpu.SemaphoreType.DMA,
        ],
    )
    def gather_kernel(table_hbm, xq_hbm, out_hbm, idxall, rowsb, transb,
                      g0, g1, w0, w1):
        gs = (g0, g1)
        ws = (w0, w1)
        wid = lax.axis_index("s") * _NUM_CORES + lax.axis_index("c")

        for st in range(seq // 8):
            pltpu.async_copy(xq_hbm.at[st, wid], idxall.at[st], w0)
        for st in range(seq // 8):
            pltpu.make_async_copy(xq_hbm.at[st, wid], idxall.at[st],
                                  w0).wait()

        def fire_gather(s, p):
            pltpu.async_copy(table_hbm.at[idxall.at[s // 8, s % 8]],
                             rowsb.at[p], gs[p])

        def wait_gather(p):
            pltpu.make_async_copy(table_hbm.at[idxall.at[0, 0]], rowsb.at[p],
                                  gs[p]).wait()

        def fire_write(s, p):
            for dt in range(d // 8):
                pltpu.async_copy(transb.at[p, pl.ds(dt * 8 * _BBLK, 8 * _BBLK)],
                                 out_hbm.at[s, dt, wid], ws[p])

        def wait_write(s, p):
            for dt in range(d // 8):
                pltpu.make_async_copy(
                    transb.at[p, pl.ds(dt * 8 * _BBLK, 8 * _BBLK)],
                    out_hbm.at[s, dt, wid], ws[p]).wait()

        iota = jnp.arange(_LANES, dtype=jnp.int32)
        rots = [jnp.mod(iota + r, _LANES) for r in range(_LANES)]
        scat = [rots[r] * _BBLK + iota for r in range(_LANES)]

        def transpose(p):
            rows = rowsb.at[p]
            trans = transb.at[p]

            def jtile(ji, carry):
                j0 = ji * _LANES
                jv = iota + j0
                for d0 in range(0, d, _LANES):
                    base = d0 * _BBLK + j0
                    for r0 in range(0, _LANES, 8):
                        dvs = [rots[r0 + q] + d0 for q in range(8)]
                        vals = [plsc.load_gather(rows, [jv, dv])
                                for dv in dvs]
                        for q, v in zip(range(8), vals):
                            plsc.store_scatter(
                                trans, [scat[r0 + q] + base], v)
                return carry

            lax.fori_loop(0, _BBLK // _LANES, jtile, 0)

        fire_gather(0, 0)
        fire_gather(1, 1)
        wait_gather(0)
        transpose(0)
        fire_write(0, 0)
        fire_gather(2, 0)
        wait_gather(1)
        transpose(1)
        fire_write(1, 1)

        def body(j, carry):
            s = 2 * j
            fire_gather(s + 1, 1)
            wait_gather(0)
            wait_write(s - 2, 0)
            transpose(0)
            fire_write(s, 0)
            fire_gather(s + 2, 0)
            wait_gather(1)
            wait_write(s - 1, 1)
            transpose(1)
            fire_write(s + 1, 1)
            return carry

        lax.fori_loop(1, seq // 2 - 1, body, 0)

        s = seq - 2
        fire_gather(s + 1, 1)
        wait_gather(0)
        wait_write(s - 2, 0)
        transpose(0)
        fire_write(s, 0)
        wait_gather(1)
        wait_write(s - 1, 1)
        transpose(1)
        fire_write(s + 1, 1)
        wait_write(s, 0)
        wait_write(s + 1, 1)

    return gather_kernel


@jax.jit
def kernel(x, table):
    batch, seq = x.shape
    vocab, d = table.shape
    xi = x.astype(jnp.int32)
    xq = jnp.transpose(xi, (1, 0)).reshape(seq // 8, 8, batch // _BBLK,
                                           _BBLK).transpose(0, 2, 1, 3)
    out4 = _make_gather(batch, seq, d, vocab)(table, xq)
    out5 = out4.reshape(seq, d // 8, batch // _BBLK, 8, _BBLK)
    return jnp.transpose(out5, (2, 4, 0, 1, 3)).reshape(batch, seq, d)


# speedup vs baseline: 1.5484x; 1.2121x over previous
import functools

import jax
import jax.numpy as jnp
from jax import lax
from jax.experimental import pallas as pl
from jax.experimental.pallas import tpu as pltpu
from jax.experimental.pallas import tpu_sc as plsc

_NUM_CORES = 2
_NUM_SUBCORES = 16
_NUM_WORKERS = _NUM_CORES * _NUM_SUBCORES
_LANES = 16
_BBLK = 128


@functools.lru_cache(maxsize=None)
def _make_gather(batch, seq, d, vocab):
    assert batch == _NUM_WORKERS * _BBLK and d % 8 == 0 and seq % 8 == 0
    mesh = plsc.VectorSubcoreMesh(core_axis_name="c", subcore_axis_name="s")

    @functools.partial(
        pl.kernel,
        mesh=mesh,
        compiler_params=pltpu.CompilerParams(use_tc_tiling_on_sc=False,
                                             needs_layout_passes=False),
        out_type=jax.ShapeDtypeStruct(
            (seq, d // 8, batch // _BBLK, 8 * _BBLK), jnp.float32),
        scratch_types=[
            pltpu.VMEM((seq // 8, 8, _BBLK), jnp.int32),
            pltpu.VMEM((4, _BBLK, 64), jnp.float32),
            pltpu.VMEM((2, d * _BBLK), jnp.float32),
            pltpu.SemaphoreType.DMA,
            pltpu.SemaphoreType.DMA,
            pltpu.SemaphoreType.DMA,
            pltpu.SemaphoreType.DMA,
            pltpu.SemaphoreType.DMA,
            pltpu.SemaphoreType.DMA,
        ],
    )
    def gather_kernel(table_hbm, xq_hbm, out_hbm, idxall, rowsb, transb,
                      g0, g1, g2, g3, w0, w1):
        gs = (g0, g1, g2, g3)
        ws = (w0, w1)
        wid = lax.axis_index("s") * _NUM_CORES + lax.axis_index("c")

        for st in range(seq // 8):
            pltpu.async_copy(xq_hbm.at[st, wid], idxall.at[st], w0)
        for st in range(seq // 8):
            pltpu.make_async_copy(xq_hbm.at[st, wid], idxall.at[st],
                                  w0).wait()

        def fire_gather(s, p):
            pltpu.async_copy(table_hbm.at[idxall.at[s // 8, s % 8]],
                             rowsb.at[p], gs[p])

        def wait_gather(p):
            pltpu.make_async_copy(table_hbm.at[idxall.at[0, 0]], rowsb.at[p],
                                  gs[p]).wait()

        def fire_write(s, p):
            for dt in range(d // 8):
                pltpu.async_copy(transb.at[p, pl.ds(dt * 8 * _BBLK, 8 * _BBLK)],
                                 out_hbm.at[s, dt, wid], ws[p])

        def wait_write(s, p):
            for dt in range(d // 8):
                pltpu.make_async_copy(
                    transb.at[p, pl.ds(dt * 8 * _BBLK, 8 * _BBLK)],
                    out_hbm.at[s, dt, wid], ws[p]).wait()

        iota = jnp.arange(_LANES, dtype=jnp.int32)
        rots = [jnp.mod(iota + r, _LANES) for r in range(_LANES)]
        scat = [rots[r] * _BBLK + iota for r in range(_LANES)]

        def transpose(b, p):
            rows = rowsb.at[b]
            trans = transb.at[p]

            def jtile(ji, carry):
                j0 = ji * _LANES
                jv = iota + j0
                for d0 in range(0, d, _LANES):
                    base = d0 * _BBLK + j0
                    for r0 in range(0, _LANES, 8):
                        dvs = [rots[r0 + q] + d0 for q in range(8)]
                        vals = [plsc.load_gather(rows, [jv, dv])
                                for dv in dvs]
                        for q, v in zip(range(8), vals):
                            plsc.store_scatter(
                                trans, [scat[r0 + q] + base], v)
                return carry

            lax.fori_loop(0, _BBLK // _LANES, jtile, 0)

        fire_gather(0, 0)
        fire_gather(1, 1)
        fire_gather(2, 2)

        def slot(s, b, p, fire, drain):
            if fire:
                fire_gather(s + 3, (b + 3) % 4)
            wait_gather(b)
            if drain:
                wait_write(s - 2, p)
            transpose_rt(b, p)
            fire_write(s, p)

        def transpose_rt(b, p):
            transpose(b, p)

        slot(0, 0, 0, True, False)
        slot(1, 1, 1, True, False)
        slot(2, 2, 0, True, True)
        slot(3, 3, 1, True, True)

        def body(j, carry):
            s = 4 * j
            slot(s, 0, 0, True, True)
            slot(s + 1, 1, 1, True, True)
            slot(s + 2, 2, 0, True, True)
            slot(s + 3, 3, 1, True, True)
            return carry

        lax.fori_loop(1, seq // 4 - 1, body, 0)

        s = seq - 4
        slot(s, 0, 0, True, True)
        slot(s + 1, 1, 1, False, True)
        slot(s + 2, 2, 0, False, True)
        slot(s + 3, 3, 1, False, True)
        wait_write(seq - 2, 0)
        wait_write(seq - 1, 1)

    return gather_kernel


@jax.jit
def kernel(x, table):
    batch, seq = x.shape
    vocab, d = table.shape
    xi = x.astype(jnp.int32)
    xq = jnp.transpose(xi, (1, 0)).reshape(seq // 8, 8, batch // _BBLK,
                                           _BBLK).transpose(0, 2, 1, 3)
    out4 = _make_gather(batch, seq, d, vocab)(table, xq)
    out5 = out4.reshape(seq, d // 8, batch // _BBLK, 8, _BBLK)
    return jnp.transpose(out5, (2, 4, 0, 1, 3)).reshape(batch, seq, d)
